# baseline (speedup 1.0000x reference)
import functools

import jax
import jax.numpy as jnp
from jax import lax
from jax.experimental import pallas as pl
from jax.experimental.pallas import tpu as pltpu
from jax.experimental.pallas import tpu_sc as plsc

NN = 10000
NP = 10240
EE = 320000
F_IN = 128
H1 = 128
GDIM = 64
BB = 64
IMG = 1280
BN_EPS = 1e-5

NC = 2
NS = 16
NW = NC * NS
EBLK = 128
BLKS_PER_TILE = 80
EPT = BLKS_PER_TILE * EBLK
EP = EPT * NW
NBUF = 2
ROWS_PER_TILE = NP // NS
RCHUNK = ROWS_PER_TILE // 4

_MESH = plsc.VectorSubcoreMesh(
    core_axis_name="c", subcore_axis_name="s", num_cores=NC, num_subcores=NS)

_F32 = jnp.float32
_PREC = lax.Precision.HIGHEST
_SC_PARAMS = pltpu.CompilerParams(use_tc_tiling_on_sc=False)


def _zero_stage(stg_v, dcols):
    zeros16 = jnp.zeros((16,), _F32)

    @pl.loop(0, RCHUNK)
    def _(i):
        for k in range(dcols // 16):
            stg_v[i, pl.ds(k * 16, 16)] = zeros16


@functools.partial(
    pl.kernel,
    out_type=jax.ShapeDtypeStruct((NC, NP, 16), _F32),
    mesh=_MESH,
    scratch_types=[
        pltpu.VMEM((NBUF, EBLK), jnp.int32),
        pltpu.VMEM((EBLK, 16), _F32),
        pltpu.VMEM((RCHUNK, 16), _F32),
        pltpu.VMEM_SHARED((NP, 16), _F32),
        pltpu.SemaphoreType.DMA,
        pltpu.SemaphoreType.DMA,
    ],
    compiler_params=_SC_PARAMS,
)
def _deg_kernel(dst_hbm, out_hbm, dst_v, ones_v, stg_v, acc, sem0, sem1):
    cid = lax.axis_index("c")
    sid = lax.axis_index("s")
    wid = cid * NS + sid
    ones16 = jnp.ones((16,), _F32)
    sems = (sem0, sem1)

    def load_idx(b, blk):
        off = pl.multiple_of(blk * EBLK, EBLK)
        pltpu.async_copy(dst_hbm.at[pl.ds(off, EBLK)], dst_v.at[b], sems[b])

    def drain_idx(b):
        pltpu.make_async_copy(
            dst_hbm.at[pl.ds(0, EBLK)], dst_v.at[b], sems[b]).wait()

    @pl.loop(0, EBLK)
    def _(i):
        ones_v[i] = ones16

    _zero_stage(stg_v, 16)
    for c in range(4):
        pltpu.sync_copy(
            stg_v, acc.at[pl.ds(sid * ROWS_PER_TILE + c * RCHUNK, RCHUNK)])
    plsc.subcore_barrier()

    base = wid * BLKS_PER_TILE
    for b in range(NBUF):
        load_idx(b, base + b)

    @pl.loop(0, BLKS_PER_TILE - NBUF, step=NBUF)
    def _(i):
        for b in range(NBUF):
            drain_idx(b)
            pltpu.sync_copy(ones_v, acc.at[dst_v.at[b]], add=True)
            load_idx(b, base + i + b + NBUF)

    for b in range(NBUF):
        drain_idx(b)
        pltpu.sync_copy(ones_v, acc.at[dst_v.at[b]], add=True)

    plsc.subcore_barrier()
    for c in range(4):
        r0 = sid * ROWS_PER_TILE + c * RCHUNK
        pltpu.sync_copy(acc.at[pl.ds(r0, RCHUNK)], stg_v)
        pltpu.sync_copy(stg_v, out_hbm.at[cid].at[pl.ds(r0, RCHUNK)])


def _make_agg(dcols):
    @functools.partial(
        pl.kernel,
        out_type=jax.ShapeDtypeStruct((NC, NP, dcols), _F32),
        mesh=_MESH,
        scratch_types=[
            pltpu.VMEM((4, 2, EBLK), jnp.int32),
            pltpu.VMEM((NBUF, EBLK, dcols), _F32),
            pltpu.VMEM_SHARED((NP, dcols), _F32),
            [pltpu.SemaphoreType.DMA] * 4,
            [pltpu.SemaphoreType.DMA] * 2,
            [pltpu.SemaphoreType.DMA] * 2,
        ],
        compiler_params=_SC_PARAMS,
    )
    def agg(hs_hbm, idx3_hbm, out_hbm, idx_v, rows_v, acc, isem, gsem, ssem):
        cid = lax.axis_index("c")
        sid = lax.axis_index("s")
        wid = cid * NS + sid
        base = wid * BLKS_PER_TILE
        LAST = BLKS_PER_TILE - 1

        def fire_idx(q, blk):
            pltpu.async_copy(idx3_hbm.at[blk], idx_v.at[q], isem[q])

        def drain_idx(q):
            pltpu.make_async_copy(
                idx3_hbm.at[base], idx_v.at[q], isem[q]).wait()

        def fire_gather(g, q):
            pltpu.async_copy(
                hs_hbm.at[idx_v.at[q, 0]], rows_v.at[g], gsem[g])

        def drain_gather(g, q):
            pltpu.make_async_copy(
                hs_hbm.at[idx_v.at[q, 0]], rows_v.at[g], gsem[g]).wait()

        def fire_scat(g, q):
            pltpu.async_copy(
                rows_v.at[g], acc.at[idx_v.at[q, 1]], ssem[g], add=True)

        def drain_scat(g, q):
            pltpu.make_async_copy(
                rows_v.at[g], acc.at[idx_v.at[q, 1]], ssem[g]).wait()

        def turn(jmod, blk, first=False, idx_on=True, gat_on=True):
            g, q = jmod % 2, jmod % 4
            drain_gather(g, q)
            fire_scat(g, q)
            if not first:
                drain_scat(g ^ 1, (jmod - 1) % 4)
            if idx_on:
                fire_idx((jmod + 3) % 4, blk + 3)
            if gat_on:
                drain_idx((jmod + 1) % 4)
                fire_gather(g ^ 1, (jmod + 1) % 4)

        zeros16 = jnp.zeros((16,), _F32)

        @pl.loop(0, EBLK)
        def _(i):
            for k in range(dcols // 16):
                rows_v[0, i, pl.ds(k * 16, 16)] = zeros16

        for c in range(ROWS_PER_TILE // EBLK):
            pltpu.sync_copy(
                rows_v.at[0],
                acc.at[pl.ds(sid * ROWS_PER_TILE + c * EBLK, EBLK)])
        plsc.subcore_barrier()

        for q in range(3):
            fire_idx(q, base + q)
        drain_idx(0)
        fire_gather(0, 0)
        turn(0, base, first=True)

        @pl.loop(0, (BLKS_PER_TILE - 8) // 4)
        def _(i):
            for t in range(4):
                turn(1 + t, base + 4 * i + 1 + t)

        for j in range(BLKS_PER_TILE - 7, BLKS_PER_TILE):
            turn(j, base + j,
                 idx_on=(j + 3 <= LAST), gat_on=(j + 1 <= LAST))
        drain_scat(LAST % 2, LAST % 4)

        plsc.subcore_barrier()
        for c in range(ROWS_PER_TILE // EBLK):
            r0 = sid * ROWS_PER_TILE + c * EBLK
            pltpu.sync_copy(acc.at[pl.ds(r0, EBLK)], rows_v.at[0])
            pltpu.sync_copy(rows_v.at[0], out_hbm.at[cid].at[pl.ds(r0, EBLK)])

    return agg


_agg128 = _make_agg(H1)
_agg64 = _make_agg(GDIM)


def _dinv_from(deg_ref):
    deg = deg_ref[0][:, 0:1] + deg_ref[1][:, 0:1] + 1.0
    return lax.rsqrt(deg)


def _scale1_body(deg_ref, x_ref, w1_ref, hs_ref):
    dinv = _dinv_from(deg_ref)
    h = jnp.dot(x_ref[...], w1_ref[...], precision=_PREC,
                preferred_element_type=_F32)
    hs_ref[...] = h * dinv


def _layer2_body(p_ref, hs1_ref, deg_ref, w2_ref, b1_ref, hs2_ref):
    dinv = _dinv_from(deg_ref)
    agg = p_ref[0] + p_ref[1] + hs1_ref[...]
    out1 = jnp.maximum(agg * dinv + b1_ref[...], 0.0)
    h2 = jnp.dot(out1, w2_ref[...], precision=_PREC,
                 preferred_element_type=_F32)
    hs2_ref[...] = h2 * dinv


def _head_body(q_ref, hs2_ref, deg_ref, b2_ref, batch_ref, img_ref,
               wi1_ref, bi1_ref, wi2_ref, bi2_ref, wc1_ref, bc1_ref,
               gamma_ref, beta_ref, wc2_ref, bc2_ref, out_ref):
    dinv = _dinv_from(deg_ref)
    out2 = (q_ref[0] + q_ref[1] + hs2_ref[...]) * dinv + b2_ref[...]
    iota = lax.broadcasted_iota(jnp.int32, (NP, BB), 1)
    oh = (batch_ref[...] == iota).astype(_F32)
    sums = lax.dot_general(oh, out2, (((0,), (0,)), ((), ())),
                           precision=_PREC, preferred_element_type=_F32)
    counts = jnp.sum(oh, axis=0)[:, None]
    ge = sums / jnp.maximum(counts, 1.0)
    img = jnp.maximum(
        jnp.dot(img_ref[...], wi1_ref[...], precision=_PREC,
                preferred_element_type=_F32) + bi1_ref[...], 0.0)
    ie = jnp.dot(img, wi2_ref[...], precision=_PREC,
                 preferred_element_type=_F32) + bi2_ref[...]
    comb = jnp.concatenate([ge, ie], axis=1)
    z = jnp.maximum(
        jnp.dot(comb, wc1_ref[...], precision=_PREC,
                preferred_element_type=_F32) + bc1_ref[...], 0.0)
    z = z * (gamma_ref[...] * (1.0 / (1.0 + BN_EPS) ** 0.5)) + beta_ref[...]
    out_ref[...] = jnp.dot(z, wc2_ref[...], precision=_PREC,
                           preferred_element_type=_F32) + bc2_ref[...]


_scale1 = pl.pallas_call(
    _scale1_body, out_shape=jax.ShapeDtypeStruct((NP, H1), _F32))
_layer2 = pl.pallas_call(
    _layer2_body, out_shape=jax.ShapeDtypeStruct((NP, GDIM), _F32))
_head = pl.pallas_call(
    _head_body, out_shape=jax.ShapeDtypeStruct((BB, 1), _F32))


def kernel(x, edge_index, batch, image_features, W1, b1, W2, b2,
           Wi1, bi1, Wi2, bi2, Wc1, bc1, gamma, beta, Wc2, bc2):
    x_pad = jnp.zeros((NP, F_IN), _F32).at[:NN].set(x)
    fill = jnp.full((EP - EE,), NN, jnp.int32)
    src_p = jnp.concatenate([edge_index[0], fill])
    dst_p = jnp.concatenate([edge_index[1], fill])
    batch_p = jnp.concatenate(
        [batch, jnp.full((NP - NN,), BB, jnp.int32)])[:, None]

    idx3 = jnp.stack(
        [src_p.reshape(-1, EBLK), dst_p.reshape(-1, EBLK)], axis=1)

    degp = _deg_kernel(dst_p)
    hs1 = _scale1(degp, x_pad, W1)
    p1 = _agg128(hs1, idx3)
    hs2 = _layer2(p1, hs1, degp, W2, b1[None, :])
    p2 = _agg64(hs2, idx3)
    out = _head(p2, hs2, degp, b2[None, :], batch_p, image_features,
                Wi1, bi1[None, :], Wi2, bi2[None, :], Wc1, bc1[None, :],
                gamma[None, :], beta[None, :], Wc2, bc2[None, :])
    return out


# speedup vs baseline: 2.5320x; 2.5320x over previous
import functools

import jax
import jax.numpy as jnp
from jax import lax
from jax.experimental import pallas as pl
from jax.experimental.pallas import tpu as pltpu
from jax.experimental.pallas import tpu_sc as plsc

NN = 10000
NP = 10240
EE = 320000
F_IN = 128
H1 = 128
GDIM = 64
BB = 64
IMG = 1280
BN_EPS = 1e-5

NC = 2
NS = 16
NW = NC * NS
EBLK = 128
BLKS_PER_TILE = 80
EPT = BLKS_PER_TILE * EBLK
EP = EPT * NW
NBUF = 2
ROWS_PER_TILE = NP // NS
RCHUNK = ROWS_PER_TILE // 4

_MESH = plsc.VectorSubcoreMesh(
    core_axis_name="c", subcore_axis_name="s", num_cores=NC, num_subcores=NS)

_F32 = jnp.float32
_PREC = lax.Precision.HIGHEST
_SC_PARAMS = pltpu.CompilerParams(use_tc_tiling_on_sc=False)


def _zero_stage(stg_v, dcols):
    zeros16 = jnp.zeros((16,), _F32)

    @pl.loop(0, RCHUNK)
    def _(i):
        for k in range(dcols // 16):
            stg_v[i, pl.ds(k * 16, 16)] = zeros16


@functools.partial(
    pl.kernel,
    out_type=jax.ShapeDtypeStruct((NC, NP, 16), _F32),
    mesh=_MESH,
    scratch_types=[
        pltpu.VMEM((NBUF, EBLK), jnp.int32),
        pltpu.VMEM((EBLK, 16), _F32),
        pltpu.VMEM((RCHUNK, 16), _F32),
        pltpu.VMEM_SHARED((NP, 16), _F32),
        pltpu.SemaphoreType.DMA,
        pltpu.SemaphoreType.DMA,
    ],
    compiler_params=_SC_PARAMS,
)
def _deg_kernel(dst_hbm, out_hbm, dst_v, ones_v, stg_v, acc, sem0, sem1):
    cid = lax.axis_index("c")
    sid = lax.axis_index("s")
    wid = cid * NS + sid
    ones16 = jnp.ones((16,), _F32)
    sems = (sem0, sem1)

    def load_idx(b, blk):
        off = pl.multiple_of(blk * EBLK, EBLK)
        pltpu.async_copy(dst_hbm.at[pl.ds(off, EBLK)], dst_v.at[b], sems[b])

    def drain_idx(b):
        pltpu.make_async_copy(
            dst_hbm.at[pl.ds(0, EBLK)], dst_v.at[b], sems[b]).wait()

    @pl.loop(0, EBLK)
    def _(i):
        ones_v[i] = ones16

    _zero_stage(stg_v, 16)
    for c in range(4):
        pltpu.sync_copy(
            stg_v, acc.at[pl.ds(sid * ROWS_PER_TILE + c * RCHUNK, RCHUNK)])
    plsc.subcore_barrier()

    base = wid * BLKS_PER_TILE
    for b in range(NBUF):
        load_idx(b, base + b)

    @pl.loop(0, BLKS_PER_TILE - NBUF, step=NBUF)
    def _(i):
        for b in range(NBUF):
            drain_idx(b)
            pltpu.sync_copy(ones_v, acc.at[dst_v.at[b]], add=True)
            load_idx(b, base + i + b + NBUF)

    for b in range(NBUF):
        drain_idx(b)
        pltpu.sync_copy(ones_v, acc.at[dst_v.at[b]], add=True)

    plsc.subcore_barrier()
    for c in range(4):
        r0 = sid * ROWS_PER_TILE + c * RCHUNK
        pltpu.sync_copy(acc.at[pl.ds(r0, RCHUNK)], stg_v)
        pltpu.sync_copy(stg_v, out_hbm.at[cid].at[pl.ds(r0, RCHUNK)])


def _make_agg(dcols):
    @functools.partial(
        pl.kernel,
        out_type=jax.ShapeDtypeStruct((NC, NP, dcols), _F32),
        mesh=_MESH,
        scratch_types=[
            pltpu.VMEM((4, 2, EBLK), jnp.int32),
            pltpu.VMEM((NBUF, EBLK, dcols), _F32),
            pltpu.VMEM_SHARED((NP, dcols), _F32),
            [pltpu.SemaphoreType.DMA] * 4,
            [pltpu.SemaphoreType.DMA] * 2,
            [pltpu.SemaphoreType.DMA] * 2,
        ],
        compiler_params=_SC_PARAMS,
    )
    def agg(hs_hbm, idx3_hbm, out_hbm, idx_v, rows_v, acc, isem, gsem, ssem):
        cid = lax.axis_index("c")
        sid = lax.axis_index("s")
        wid = cid * NS + sid
        base = wid * BLKS_PER_TILE
        LAST = BLKS_PER_TILE - 1

        def fire_idx(q, blk):
            pltpu.async_copy(idx3_hbm.at[blk], idx_v.at[q], isem[q])

        def drain_idx(q):
            pltpu.make_async_copy(
                idx3_hbm.at[base], idx_v.at[q], isem[q]).wait()

        def fire_gather(g, q):
            pltpu.async_copy(
                hs_hbm.at[idx_v.at[q, 0]], rows_v.at[g], gsem[g])

        def drain_gather(g, q):
            pltpu.make_async_copy(
                hs_hbm.at[idx_v.at[q, 0]], rows_v.at[g], gsem[g]).wait()

        def fire_scat(g, q):
            pltpu.async_copy(
                rows_v.at[g], acc.at[idx_v.at[q, 1]], ssem[g], add=True)

        def drain_scat(g, q):
            pltpu.make_async_copy(
                rows_v.at[g], acc.at[idx_v.at[q, 1]], ssem[g]).wait()

        def turn(jmod, blk, first=False, idx_on=True, gat_on=True):
            g, q = jmod % 2, jmod % 4
            drain_gather(g, q)
            fire_scat(g, q)
            if not first:
                drain_scat(g ^ 1, (jmod - 1) % 4)
            if idx_on:
                fire_idx((jmod + 3) % 4, blk + 3)
            if gat_on:
                drain_idx((jmod + 1) % 4)
                fire_gather(g ^ 1, (jmod + 1) % 4)

        zeros16 = jnp.zeros((16,), _F32)

        @pl.loop(0, EBLK)
        def _(i):
            for k in range(dcols // 16):
                rows_v[0, i, pl.ds(k * 16, 16)] = zeros16

        for c in range(ROWS_PER_TILE // EBLK):
            pltpu.sync_copy(
                rows_v.at[0],
                acc.at[pl.ds(sid * ROWS_PER_TILE + c * EBLK, EBLK)])
        plsc.subcore_barrier()

        for q in range(3):
            fire_idx(q, base + q)
        drain_idx(0)
        fire_gather(0, 0)
        turn(0, base, first=True)

        @pl.loop(0, (BLKS_PER_TILE - 8) // 4)
        def _(i):
            for t in range(4):
                turn(1 + t, base + 4 * i + 1 + t)

        for j in range(BLKS_PER_TILE - 7, BLKS_PER_TILE):
            turn(j, base + j,
                 idx_on=(j + 3 <= LAST), gat_on=(j + 1 <= LAST))
        drain_scat(LAST % 2, LAST % 4)

        plsc.subcore_barrier()
        for c in range(ROWS_PER_TILE // EBLK):
            r0 = sid * ROWS_PER_TILE + c * EBLK
            pltpu.sync_copy(acc.at[pl.ds(r0, EBLK)], rows_v.at[0])
            pltpu.sync_copy(rows_v.at[0], out_hbm.at[cid].at[pl.ds(r0, EBLK)])

    return agg


_agg128 = _make_agg(H1)
_agg64 = _make_agg(GDIM)


def _dinv_from(deg_ref):
    deg = deg_ref[0][:, 0:1] + deg_ref[1][:, 0:1] + 1.0
    return lax.rsqrt(deg)


def _scale1_body(deg_ref, x_ref, w1_ref, hs_ref):
    dinv = _dinv_from(deg_ref)
    h = jnp.dot(x_ref[...], w1_ref[...], precision=_PREC,
                preferred_element_type=_F32)
    hs_ref[...] = h * dinv


def _layer2_body(p_ref, hs1_ref, deg_ref, w2_ref, b1_ref, hs2_ref):
    dinv = _dinv_from(deg_ref)
    agg = p_ref[0] + p_ref[1] + hs1_ref[...]
    out1 = jnp.maximum(agg * dinv + b1_ref[...], 0.0)
    h2 = jnp.dot(out1, w2_ref[...], precision=_PREC,
                 preferred_element_type=_F32)
    hs2_ref[...] = h2 * dinv


def _head_body(q_ref, hs2_ref, deg_ref, b2_ref, batch_ref, img_ref,
               wi1_ref, bi1_ref, wi2_ref, bi2_ref, wc1_ref, bc1_ref,
               gamma_ref, beta_ref, wc2_ref, bc2_ref, out_ref):
    dinv = _dinv_from(deg_ref)
    out2 = (q_ref[0] + q_ref[1] + hs2_ref[...]) * dinv + b2_ref[...]
    iota = lax.broadcasted_iota(jnp.int32, (NP, BB), 1)
    oh = (batch_ref[...] == iota).astype(_F32)
    sums = lax.dot_general(oh, out2, (((0,), (0,)), ((), ())),
                           precision=_PREC, preferred_element_type=_F32)
    counts = jnp.sum(oh, axis=0)[:, None]
    ge = sums / jnp.maximum(counts, 1.0)
    img = jnp.maximum(
        jnp.dot(img_ref[...], wi1_ref[...], precision=_PREC,
                preferred_element_type=_F32) + bi1_ref[...], 0.0)
    ie = jnp.dot(img, wi2_ref[...], precision=_PREC,
                 preferred_element_type=_F32) + bi2_ref[...]
    comb = jnp.concatenate([ge, ie], axis=1)
    z = jnp.maximum(
        jnp.dot(comb, wc1_ref[...], precision=_PREC,
                preferred_element_type=_F32) + bc1_ref[...], 0.0)
    z = z * (gamma_ref[...] * (1.0 / (1.0 + BN_EPS) ** 0.5)) + beta_ref[...]
    out_ref[...] = jnp.dot(z, wc2_ref[...], precision=_PREC,
                           preferred_element_type=_F32) + bc2_ref[...]


_scale1 = pl.pallas_call(
    _scale1_body, out_shape=jax.ShapeDtypeStruct((NP, H1), _F32))
_layer2 = pl.pallas_call(
    _layer2_body, out_shape=jax.ShapeDtypeStruct((NP, GDIM), _F32))
_head = pl.pallas_call(
    _head_body, out_shape=jax.ShapeDtypeStruct((BB, 1), _F32))


def kernel(x, edge_index, batch, image_features, W1, b1, W2, b2,
           Wi1, bi1, Wi2, bi2, Wc1, bc1, gamma, beta, Wc2, bc2):
    x_pad = jnp.zeros((NP, F_IN), _F32).at[:NN].set(x)
    fill = NN + (jnp.arange(EP - EE, dtype=jnp.int32) % (NP - NN))
    src_p = jnp.concatenate([edge_index[0], fill])
    dst_p = jnp.concatenate([edge_index[1], fill])
    batch_p = jnp.concatenate(
        [batch, jnp.full((NP - NN,), BB, jnp.int32)])[:, None]

    idx3 = jnp.stack(
        [src_p.reshape(-1, EBLK), dst_p.reshape(-1, EBLK)], axis=1)

    degp = _deg_kernel(dst_p)
    hs1 = _scale1(degp, x_pad, W1)
    p1 = _agg128(hs1, idx3)
    hs2 = _layer2(p1, hs1, degp, W2, b1[None, :])
    p2 = _agg64(hs2, idx3)
    out = _head(p2, hs2, degp, b2[None, :], batch_p, image_features,
                Wi1, bi1[None, :], Wi2, bi2[None, :], Wc1, bc1[None, :],
                gamma[None, :], beta[None, :], Wc2, bc2[None, :])
    return out


# speedup vs baseline: 2.5359x; 1.0016x over previous
import functools

import jax
import jax.numpy as jnp
from jax import lax
from jax.experimental import pallas as pl
from jax.experimental.pallas import tpu as pltpu
from jax.experimental.pallas import tpu_sc as plsc

NN = 10000
NP = 10240
EE = 320000
F_IN = 128
H1 = 128
GDIM = 64
BB = 64
IMG = 1280
BN_EPS = 1e-5

NC = 2
NS = 16
NW = NC * NS
EBLK = 128
BLKS_PER_TILE = 80
EPT = BLKS_PER_TILE * EBLK
EP = EPT * NW
NBUF = 2
ROWS_PER_TILE = NP // NS
RCHUNK = ROWS_PER_TILE // 4

_MESH = plsc.VectorSubcoreMesh(
    core_axis_name="c", subcore_axis_name="s", num_cores=NC, num_subcores=NS)

_F32 = jnp.float32
_PREC = lax.Precision.HIGHEST
_SC_PARAMS = pltpu.CompilerParams(use_tc_tiling_on_sc=False)


def _zero_stage(stg_v, dcols):
    zeros16 = jnp.zeros((16,), _F32)

    @pl.loop(0, RCHUNK)
    def _(i):
        for k in range(dcols // 16):
            stg_v[i, pl.ds(k * 16, 16)] = zeros16


@functools.partial(
    pl.kernel,
    out_type=jax.ShapeDtypeStruct((NC, NP, 16), _F32),
    mesh=_MESH,
    scratch_types=[
        pltpu.VMEM((NBUF, EBLK), jnp.int32),
        pltpu.VMEM((EBLK, 16), _F32),
        pltpu.VMEM((RCHUNK, 16), _F32),
        pltpu.VMEM_SHARED((NP, 16), _F32),
        pltpu.SemaphoreType.DMA,
        pltpu.SemaphoreType.DMA,
    ],
    compiler_params=_SC_PARAMS,
)
def _deg_kernel(dst_hbm, out_hbm, dst_v, ones_v, stg_v, acc, sem0, sem1):
    cid = lax.axis_index("c")
    sid = lax.axis_index("s")
    wid = cid * NS + sid
    ones16 = jnp.ones((16,), _F32)
    sems = (sem0, sem1)

    def load_idx(b, blk):
        off = pl.multiple_of(blk * EBLK, EBLK)
        pltpu.async_copy(dst_hbm.at[pl.ds(off, EBLK)], dst_v.at[b], sems[b])

    def drain_idx(b):
        pltpu.make_async_copy(
            dst_hbm.at[pl.ds(0, EBLK)], dst_v.at[b], sems[b]).wait()

    @pl.loop(0, EBLK)
    def _(i):
        ones_v[i] = ones16

    _zero_stage(stg_v, 16)
    for c in range(4):
        pltpu.sync_copy(
            stg_v, acc.at[pl.ds(sid * ROWS_PER_TILE + c * RCHUNK, RCHUNK)])
    plsc.subcore_barrier()

    base = wid * BLKS_PER_TILE
    for b in range(NBUF):
        load_idx(b, base + b)

    @pl.loop(0, BLKS_PER_TILE - NBUF, step=NBUF)
    def _(i):
        for b in range(NBUF):
            drain_idx(b)
            pltpu.sync_copy(ones_v, acc.at[dst_v.at[b]], add=True)
            load_idx(b, base + i + b + NBUF)

    for b in range(NBUF):
        drain_idx(b)
        pltpu.sync_copy(ones_v, acc.at[dst_v.at[b]], add=True)

    plsc.subcore_barrier()
    for c in range(4):
        r0 = sid * ROWS_PER_TILE + c * RCHUNK
        pltpu.sync_copy(acc.at[pl.ds(r0, RCHUNK)], stg_v)
        pltpu.sync_copy(stg_v, out_hbm.at[cid].at[pl.ds(r0, RCHUNK)])


def _make_agg(dcols, nr):
    nq = 2 * nr
    pf = nr + 1

    @functools.partial(
        pl.kernel,
        out_type=jax.ShapeDtypeStruct((NC, NP, dcols), _F32),
        mesh=_MESH,
        scratch_types=[
            pltpu.VMEM((nq, 2, EBLK), jnp.int32),
            pltpu.VMEM((nr, EBLK, dcols), _F32),
            pltpu.VMEM_SHARED((NP, dcols), _F32),
            [pltpu.SemaphoreType.DMA] * nq,
            [pltpu.SemaphoreType.DMA] * nr,
            [pltpu.SemaphoreType.DMA] * nr,
        ],
        compiler_params=_SC_PARAMS,
    )
    def agg(hs_hbm, idx3_hbm, out_hbm, idx_v, rows_v, acc, isem, gsem, ssem):
        cid = lax.axis_index("c")
        sid = lax.axis_index("s")
        wid = cid * NS + sid
        base = wid * BLKS_PER_TILE
        LAST = BLKS_PER_TILE - 1

        def fire_idx(q, blk):
            pltpu.async_copy(idx3_hbm.at[blk], idx_v.at[q], isem[q])

        def drain_idx(q):
            pltpu.make_async_copy(
                idx3_hbm.at[base], idx_v.at[q], isem[q]).wait()

        def fire_gather(g, q):
            pltpu.async_copy(
                hs_hbm.at[idx_v.at[q, 0]], rows_v.at[g], gsem[g])

        def drain_gather(g, q):
            pltpu.make_async_copy(
                hs_hbm.at[idx_v.at[q, 0]], rows_v.at[g], gsem[g]).wait()

        def fire_scat(g, q):
            pltpu.async_copy(
                rows_v.at[g], acc.at[idx_v.at[q, 1]], ssem[g], add=True)

        def drain_scat(g, q):
            pltpu.make_async_copy(
                rows_v.at[g], acc.at[idx_v.at[q, 1]], ssem[g]).wait()

        def turn(jmod, blk, scat_on=True, idx_on=True, gat_on=True):
            g, q = jmod % nr, jmod % nq
            drain_gather(g, q)
            fire_scat(g, q)
            if scat_on:
                drain_scat((jmod + 1) % nr, (jmod + 1 - nr) % nq)
            if idx_on:
                fire_idx((jmod + pf) % nq, blk + pf)
            if gat_on:
                drain_idx((jmod + 1) % nq)
                fire_gather((jmod + 1) % nr, (jmod + 1) % nq)

        zeros16 = jnp.zeros((16,), _F32)

        @pl.loop(0, EBLK)
        def _(i):
            for k in range(dcols // 16):
                rows_v[0, i, pl.ds(k * 16, 16)] = zeros16

        for c in range(ROWS_PER_TILE // EBLK):
            pltpu.sync_copy(
                rows_v.at[0],
                acc.at[pl.ds(sid * ROWS_PER_TILE + c * EBLK, EBLK)])
        plsc.subcore_barrier()

        for q in range(pf):
            fire_idx(q, base + q)
        drain_idx(0)
        fire_gather(0, 0)
        for j in range(nq):
            turn(j, base + j, scat_on=(j + 1 - nr >= 0))

        @pl.loop(0, (BLKS_PER_TILE - 2 * nq) // nq)
        def _(i):
            for t in range(nq):
                turn(t, base + nq + nq * i + t)

        for j in range(BLKS_PER_TILE - nq, BLKS_PER_TILE):
            turn(j, base + j,
                 idx_on=(j + pf <= LAST), gat_on=(j + 1 <= LAST))
        for j in range(BLKS_PER_TILE - nr + 1, BLKS_PER_TILE):
            drain_scat(j % nr, j % nq)

        plsc.subcore_barrier()
        for c in range(ROWS_PER_TILE // EBLK):
            r0 = sid * ROWS_PER_TILE + c * EBLK
            pltpu.sync_copy(acc.at[pl.ds(r0, EBLK)], rows_v.at[0])
            pltpu.sync_copy(rows_v.at[0], out_hbm.at[cid].at[pl.ds(r0, EBLK)])

    return agg


_agg128 = _make_agg(H1, 2)
_agg64 = _make_agg(GDIM, 4)


def _dinv_from(deg_ref):
    deg = (deg_ref[0, pl.ds(0, NN), 0:1] + deg_ref[1, pl.ds(0, NN), 0:1]
           + 1.0)
    return lax.rsqrt(deg)


def _scale1_body(deg_ref, x_ref, w1_ref, hs_ref):
    dinv = _dinv_from(deg_ref)
    h = jnp.dot(x_ref[...], w1_ref[...], precision=_PREC,
                preferred_element_type=_F32)
    hs_ref[pl.ds(0, NN), :] = h * dinv
    hs_ref[pl.ds(NN, NP - NN), :] = jnp.zeros((NP - NN, H1), _F32)


def _layer2_body(p_ref, hs1_ref, deg_ref, w2_ref, b1_ref, hs2_ref):
    dinv = _dinv_from(deg_ref)
    agg = (p_ref[0, pl.ds(0, NN), :] + p_ref[1, pl.ds(0, NN), :]
           + hs1_ref[pl.ds(0, NN), :])
    out1 = jnp.maximum(agg * dinv + b1_ref[...], 0.0)
    h2 = jnp.dot(out1, w2_ref[...], precision=_PREC,
                 preferred_element_type=_F32)
    hs2_ref[pl.ds(0, NN), :] = h2 * dinv
    hs2_ref[pl.ds(NN, NP - NN), :] = jnp.zeros((NP - NN, GDIM), _F32)


def _head_body(q_ref, hs2_ref, deg_ref, b2_ref, batch_ref, img_ref,
               wi1_ref, bi1_ref, wi2_ref, bi2_ref, wc1_ref, bc1_ref,
               gamma_ref, beta_ref, wc2_ref, bc2_ref, out_ref):
    dinv = _dinv_from(deg_ref)
    out2 = ((q_ref[0, pl.ds(0, NN), :] + q_ref[1, pl.ds(0, NN), :]
             + hs2_ref[pl.ds(0, NN), :]) * dinv + b2_ref[...])
    iota = lax.broadcasted_iota(jnp.int32, (NN, BB), 1)
    oh = (batch_ref[...] == iota).astype(_F32)
    sums = lax.dot_general(oh, out2, (((0,), (0,)), ((), ())),
                           precision=_PREC, preferred_element_type=_F32)
    counts = jnp.sum(oh, axis=0)[:, None]
    ge = sums / jnp.maximum(counts, 1.0)
    img = jnp.maximum(
        jnp.dot(img_ref[...], wi1_ref[...], precision=_PREC,
                preferred_element_type=_F32) + bi1_ref[...], 0.0)
    ie = jnp.dot(img, wi2_ref[...], precision=_PREC,
                 preferred_element_type=_F32) + bi2_ref[...]
    comb = jnp.concatenate([ge, ie], axis=1)
    z = jnp.maximum(
        jnp.dot(comb, wc1_ref[...], precision=_PREC,
                preferred_element_type=_F32) + bc1_ref[...], 0.0)
    z = z * (gamma_ref[...] * (1.0 / (1.0 + BN_EPS) ** 0.5)) + beta_ref[...]
    out_ref[...] = jnp.dot(z, wc2_ref[...], precision=_PREC,
                           preferred_element_type=_F32) + bc2_ref[...]


_scale1 = pl.pallas_call(
    _scale1_body, out_shape=jax.ShapeDtypeStruct((NP, H1), _F32))
_layer2 = pl.pallas_call(
    _layer2_body, out_shape=jax.ShapeDtypeStruct((NP, GDIM), _F32))
_head = pl.pallas_call(
    _head_body, out_shape=jax.ShapeDtypeStruct((BB, 1), _F32))


def kernel(x, edge_index, batch, image_features, W1, b1, W2, b2,
           Wi1, bi1, Wi2, bi2, Wc1, bc1, gamma, beta, Wc2, bc2):
    fill = NN + (jnp.arange(EP - EE, dtype=jnp.int32) % (NP - NN))
    src_p = jnp.concatenate([edge_index[0], fill])
    dst_p = jnp.concatenate([edge_index[1], fill])

    idx3 = jnp.stack(
        [src_p.reshape(-1, EBLK), dst_p.reshape(-1, EBLK)], axis=1)

    degp = _deg_kernel(dst_p)
    hs1 = _scale1(degp, x, W1)
    p1 = _agg128(hs1, idx3)
    hs2 = _layer2(p1, hs1, degp, W2, b1[None, :])
    p2 = _agg64(hs2, idx3)
    out = _head(p2, hs2, degp, b2[None, :], batch[:, None], image_features,
                Wi1, bi1[None, :], Wi2, bi2[None, :], Wc1, bc1[None, :],
                gamma[None, :], beta[None, :], Wc2, bc2[None, :])
    return out


# speedup vs baseline: 2.6081x; 1.0285x over previous
import functools

import jax
import jax.numpy as jnp
from jax import lax
from jax.experimental import pallas as pl
from jax.experimental.pallas import tpu as pltpu
from jax.experimental.pallas import tpu_sc as plsc

NN = 10000
NP = 10240
EE = 320000
F_IN = 128
H1 = 128
GDIM = 64
BB = 64
IMG = 1280
BN_EPS = 1e-5

NC = 2
NS = 16
NW = NC * NS
EBLK = 128
BLKS_PER_TILE = 80
EPT = BLKS_PER_TILE * EBLK
EP = EPT * NW
NBUF = 2
ROWS_PER_TILE = NP // NS
RCHUNK = ROWS_PER_TILE // 4

_MESH = plsc.VectorSubcoreMesh(
    core_axis_name="c", subcore_axis_name="s", num_cores=NC, num_subcores=NS)

_F32 = jnp.float32
_PREC = None
_SC_PARAMS = pltpu.CompilerParams(use_tc_tiling_on_sc=False)


def _zero_stage(stg_v, dcols):
    zeros16 = jnp.zeros((16,), _F32)

    @pl.loop(0, RCHUNK)
    def _(i):
        for k in range(dcols // 16):
            stg_v[i, pl.ds(k * 16, 16)] = zeros16


@functools.partial(
    pl.kernel,
    out_type=jax.ShapeDtypeStruct((NC, NP, 16), _F32),
    mesh=_MESH,
    scratch_types=[
        pltpu.VMEM((NBUF, EBLK), jnp.int32),
        pltpu.VMEM((EBLK, 16), _F32),
        pltpu.VMEM((RCHUNK, 16), _F32),
        pltpu.VMEM_SHARED((NP, 16), _F32),
        pltpu.SemaphoreType.DMA,
        pltpu.SemaphoreType.DMA,
    ],
    compiler_params=_SC_PARAMS,
)
def _deg_kernel(dst_hbm, out_hbm, dst_v, ones_v, stg_v, acc, sem0, sem1):
    cid = lax.axis_index("c")
    sid = lax.axis_index("s")
    wid = cid * NS + sid
    ones16 = jnp.ones((16,), _F32)
    sems = (sem0, sem1)

    def load_idx(b, blk):
        off = pl.multiple_of(blk * EBLK, EBLK)
        pltpu.async_copy(dst_hbm.at[pl.ds(off, EBLK)], dst_v.at[b], sems[b])

    def drain_idx(b):
        pltpu.make_async_copy(
            dst_hbm.at[pl.ds(0, EBLK)], dst_v.at[b], sems[b]).wait()

    @pl.loop(0, EBLK)
    def _(i):
        ones_v[i] = ones16

    _zero_stage(stg_v, 16)
    for c in range(4):
        pltpu.sync_copy(
            stg_v, acc.at[pl.ds(sid * ROWS_PER_TILE + c * RCHUNK, RCHUNK)])
    plsc.subcore_barrier()

    base = wid * BLKS_PER_TILE
    for b in range(NBUF):
        load_idx(b, base + b)

    @pl.loop(0, BLKS_PER_TILE - NBUF, step=NBUF)
    def _(i):
        for b in range(NBUF):
            drain_idx(b)
            pltpu.sync_copy(ones_v, acc.at[dst_v.at[b]], add=True)
            load_idx(b, base + i + b + NBUF)

    for b in range(NBUF):
        drain_idx(b)
        pltpu.sync_copy(ones_v, acc.at[dst_v.at[b]], add=True)

    plsc.subcore_barrier()
    for c in range(4):
        r0 = sid * ROWS_PER_TILE + c * RCHUNK
        pltpu.sync_copy(acc.at[pl.ds(r0, RCHUNK)], stg_v)
        pltpu.sync_copy(stg_v, out_hbm.at[cid].at[pl.ds(r0, RCHUNK)])


def _make_agg(dcols, nr):
    nq = 2 * nr
    pf = nr + 1

    @functools.partial(
        pl.kernel,
        out_type=jax.ShapeDtypeStruct((NC, NP, dcols), _F32),
        mesh=_MESH,
        scratch_types=[
            pltpu.VMEM((nq, 2, EBLK), jnp.int32),
            pltpu.VMEM((nr, EBLK, dcols), _F32),
            pltpu.VMEM_SHARED((NP, dcols), _F32),
            [pltpu.SemaphoreType.DMA] * nq,
            [pltpu.SemaphoreType.DMA] * nr,
            [pltpu.SemaphoreType.DMA] * nr,
        ],
        compiler_params=_SC_PARAMS,
    )
    def agg(hs_hbm, idx3_hbm, out_hbm, idx_v, rows_v, acc, isem, gsem, ssem):
        cid = lax.axis_index("c")
        sid = lax.axis_index("s")
        wid = cid * NS + sid
        base = wid * BLKS_PER_TILE
        LAST = BLKS_PER_TILE - 1

        def fire_idx(q, blk):
            pltpu.async_copy(idx3_hbm.at[blk], idx_v.at[q], isem[q])

        def drain_idx(q):
            pltpu.make_async_copy(
                idx3_hbm.at[base], idx_v.at[q], isem[q]).wait()

        def fire_gather(g, q):
            pltpu.async_copy(
                hs_hbm.at[idx_v.at[q, 0]], rows_v.at[g], gsem[g])

        def drain_gather(g, q):
            pltpu.make_async_copy(
                hs_hbm.at[idx_v.at[q, 0]], rows_v.at[g], gsem[g]).wait()

        def fire_scat(g, q):
            pltpu.async_copy(
                rows_v.at[g], acc.at[idx_v.at[q, 1]], ssem[g], add=True)

        def drain_scat(g, q):
            pltpu.make_async_copy(
                rows_v.at[g], acc.at[idx_v.at[q, 1]], ssem[g]).wait()

        def turn(jmod, blk, scat_on=True, idx_on=True, gat_on=True):
            g, q = jmod % nr, jmod % nq
            drain_gather(g, q)
            fire_scat(g, q)
            if scat_on:
                drain_scat((jmod + 1) % nr, (jmod + 1 - nr) % nq)
            if idx_on:
                fire_idx((jmod + pf) % nq, blk + pf)
            if gat_on:
                drain_idx((jmod + 1) % nq)
                fire_gather((jmod + 1) % nr, (jmod + 1) % nq)

        zeros16 = jnp.zeros((16,), _F32)

        @pl.loop(0, EBLK)
        def _(i):
            for k in range(dcols // 16):
                rows_v[0, i, pl.ds(k * 16, 16)] = zeros16

        for c in range(ROWS_PER_TILE // EBLK):
            pltpu.sync_copy(
                rows_v.at[0],
                acc.at[pl.ds(sid * ROWS_PER_TILE + c * EBLK, EBLK)])
        plsc.subcore_barrier()

        for q in range(pf):
            fire_idx(q, base + q)
        drain_idx(0)
        fire_gather(0, 0)
        for j in range(nq):
            turn(j, base + j, scat_on=(j + 1 - nr >= 0))

        @pl.loop(0, (BLKS_PER_TILE - 2 * nq) // nq)
        def _(i):
            for t in range(nq):
                turn(t, base + nq + nq * i + t)

        for j in range(BLKS_PER_TILE - nq, BLKS_PER_TILE):
            turn(j, base + j,
                 idx_on=(j + pf <= LAST), gat_on=(j + 1 <= LAST))
        for j in range(BLKS_PER_TILE - nr + 1, BLKS_PER_TILE):
            drain_scat(j % nr, j % nq)

        plsc.subcore_barrier()
        for c in range(ROWS_PER_TILE // EBLK):
            r0 = sid * ROWS_PER_TILE + c * EBLK
            pltpu.sync_copy(acc.at[pl.ds(r0, EBLK)], rows_v.at[0])
            pltpu.sync_copy(rows_v.at[0], out_hbm.at[cid].at[pl.ds(r0, EBLK)])

    return agg


_agg128 = _make_agg(H1, 2)
_agg64 = _make_agg(GDIM, 4)


def _dinv_from(deg_ref):
    deg = (deg_ref[0, pl.ds(0, NN), 0:1] + deg_ref[1, pl.ds(0, NN), 0:1]
           + 1.0)
    return lax.rsqrt(deg)


def _scale1_body(deg_ref, x_ref, w1_ref, hs_ref):
    dinv = _dinv_from(deg_ref)
    h = jnp.dot(x_ref[...], w1_ref[...], precision=_PREC,
                preferred_element_type=_F32)
    hs_ref[pl.ds(0, NN), :] = h * dinv
    hs_ref[pl.ds(NN, NP - NN), :] = jnp.zeros((NP - NN, H1), _F32)


def _layer2_body(p_ref, hs1_ref, deg_ref, w2_ref, b1_ref, hs2_ref):
    dinv = _dinv_from(deg_ref)
    agg = (p_ref[0, pl.ds(0, NN), :] + p_ref[1, pl.ds(0, NN), :]
           + hs1_ref[pl.ds(0, NN), :])
    out1 = jnp.maximum(agg * dinv + b1_ref[...], 0.0)
    h2 = jnp.dot(out1, w2_ref[...], precision=_PREC,
                 preferred_element_type=_F32)
    hs2_ref[pl.ds(0, NN), :] = h2 * dinv
    hs2_ref[pl.ds(NN, NP - NN), :] = jnp.zeros((NP - NN, GDIM), _F32)


def _head_body(q_ref, hs2_ref, deg_ref, b2_ref, batch_ref, img_ref,
               wi1_ref, bi1_ref, wi2_ref, bi2_ref, wc1_ref, bc1_ref,
               gamma_ref, beta_ref, wc2_ref, bc2_ref, out_ref):
    dinv = _dinv_from(deg_ref)
    out2 = ((q_ref[0, pl.ds(0, NN), :] + q_ref[1, pl.ds(0, NN), :]
             + hs2_ref[pl.ds(0, NN), :]) * dinv + b2_ref[...])
    iota = lax.broadcasted_iota(jnp.int32, (NN, BB), 1)
    oh = (batch_ref[...] == iota).astype(_F32)
    sums = lax.dot_general(oh, out2, (((0,), (0,)), ((), ())),
                           precision=_PREC, preferred_element_type=_F32)
    counts = jnp.sum(oh, axis=0)[:, None]
    ge = sums / jnp.maximum(counts, 1.0)
    img = jnp.maximum(
        jnp.dot(img_ref[...], wi1_ref[...], precision=_PREC,
                preferred_element_type=_F32) + bi1_ref[...], 0.0)
    ie = jnp.dot(img, wi2_ref[...], precision=_PREC,
                 preferred_element_type=_F32) + bi2_ref[...]
    comb = jnp.concatenate([ge, ie], axis=1)
    z = jnp.maximum(
        jnp.dot(comb, wc1_ref[...], precision=_PREC,
                preferred_element_type=_F32) + bc1_ref[...], 0.0)
    z = z * (gamma_ref[...] * (1.0 / (1.0 + BN_EPS) ** 0.5)) + beta_ref[...]
    out_ref[...] = jnp.dot(z, wc2_ref[...], precision=_PREC,
                           preferred_element_type=_F32) + bc2_ref[...]


_scale1 = pl.pallas_call(
    _scale1_body, out_shape=jax.ShapeDtypeStruct((NP, H1), _F32))
_layer2 = pl.pallas_call(
    _layer2_body, out_shape=jax.ShapeDtypeStruct((NP, GDIM), _F32))
_head = pl.pallas_call(
    _head_body, out_shape=jax.ShapeDtypeStruct((BB, 1), _F32))


def kernel(x, edge_index, batch, image_features, W1, b1, W2, b2,
           Wi1, bi1, Wi2, bi2, Wc1, bc1, gamma, beta, Wc2, bc2):
    fill = NN + (jnp.arange(EP - EE, dtype=jnp.int32) % (NP - NN))
    src_p = jnp.concatenate([edge_index[0], fill])
    dst_p = jnp.concatenate([edge_index[1], fill])

    idx3 = jnp.stack(
        [src_p.reshape(-1, EBLK), dst_p.reshape(-1, EBLK)], axis=1)

    degp = _deg_kernel(dst_p)
    hs1 = _scale1(degp, x, W1)
    p1 = _agg128(hs1, idx3)
    hs2 = _layer2(p1, hs1, degp, W2, b1[None, :])
    p2 = _agg64(hs2, idx3)
    out = _head(p2, hs2, degp, b2[None, :], batch[:, None], image_features,
                Wi1, bi1[None, :], Wi2, bi2[None, :], Wc1, bc1[None, :],
                gamma[None, :], beta[None, :], Wc2, bc2[None, :])
    return out


# speedup vs baseline: 2.7327x; 1.0478x over previous
import functools

import jax
import jax.numpy as jnp
from jax import lax
from jax.experimental import pallas as pl
from jax.experimental.pallas import tpu as pltpu
from jax.experimental.pallas import tpu_sc as plsc

NN = 10000
NP = 10240
EE = 320000
F_IN = 128
H1 = 128
GDIM = 64
BB = 64
IMG = 1280
BN_EPS = 1e-5

NC = 2
NS = 16
NW = NC * NS
EBLK = 128
BLKS_PER_TILE = 80
EPT = BLKS_PER_TILE * EBLK
EP = EPT * NW
NBUF = 2
ROWS_PER_TILE = NP // NS
RCHUNK = ROWS_PER_TILE // 4

_MESH = plsc.VectorSubcoreMesh(
    core_axis_name="c", subcore_axis_name="s", num_cores=NC, num_subcores=NS)

_F32 = jnp.float32
_PREC = None
_SC_PARAMS = pltpu.CompilerParams(use_tc_tiling_on_sc=False)


def _zero_stage(stg_v, dcols):
    zeros16 = jnp.zeros((16,), _F32)

    @pl.loop(0, RCHUNK)
    def _(i):
        for k in range(dcols // 16):
            stg_v[i, pl.ds(k * 16, 16)] = zeros16


@functools.partial(
    pl.kernel,
    out_type=jax.ShapeDtypeStruct((NC, NP, 16), _F32),
    mesh=_MESH,
    scratch_types=[
        pltpu.VMEM((8, EBLK), jnp.int32),
        pltpu.VMEM((EBLK, 16), _F32),
        pltpu.VMEM((RCHUNK, 16), _F32),
        pltpu.VMEM_SHARED((NP, 16), _F32),
        [pltpu.SemaphoreType.DMA] * 8,
        [pltpu.SemaphoreType.DMA] * 4,
    ],
    compiler_params=_SC_PARAMS,
)
def _deg_kernel(dst_hbm, out_hbm, dst_v, ones_v, stg_v, acc, isem, ssem):
    cid = lax.axis_index("c")
    sid = lax.axis_index("s")
    wid = cid * NS + sid
    ones16 = jnp.ones((16,), _F32)
    base = wid * BLKS_PER_TILE
    LAST = BLKS_PER_TILE - 1

    def fire_idx(q, blk):
        off = pl.multiple_of(blk * EBLK, EBLK)
        pltpu.async_copy(dst_hbm.at[pl.ds(off, EBLK)], dst_v.at[q], isem[q])

    def drain_idx(q):
        pltpu.make_async_copy(
            dst_hbm.at[pl.ds(0, EBLK)], dst_v.at[q], isem[q]).wait()

    def fire_scat(s, q):
        pltpu.async_copy(ones_v, acc.at[dst_v.at[q]], ssem[s], add=True)

    def drain_scat(s, q):
        pltpu.make_async_copy(ones_v, acc.at[dst_v.at[q]], ssem[s]).wait()

    def turn(jmod, blk, scat_on=True, idx_on=True):
        q, s = jmod % 8, jmod % 4
        drain_idx(q)
        fire_scat(s, q)
        if scat_on:
            drain_scat((jmod + 1) % 4, (jmod - 3) % 8)
        if idx_on:
            fire_idx((jmod + 5) % 8, blk + 5)

    @pl.loop(0, EBLK)
    def _(i):
        ones_v[i] = ones16

    _zero_stage(stg_v, 16)
    for c in range(4):
        pltpu.sync_copy(
            stg_v, acc.at[pl.ds(sid * ROWS_PER_TILE + c * RCHUNK, RCHUNK)])
    plsc.subcore_barrier()

    for q in range(5):
        fire_idx(q, base + q)
    for j in range(8):
        turn(j, base + j, scat_on=(j - 3 >= 0))

    @pl.loop(0, (BLKS_PER_TILE - 16) // 8)
    def _(i):
        for t in range(8):
            turn(t, base + 8 + 8 * i + t)

    for j in range(BLKS_PER_TILE - 8, BLKS_PER_TILE):
        turn(j, base + j, idx_on=(j + 5 <= LAST))
    for j in range(BLKS_PER_TILE - 3, BLKS_PER_TILE):
        drain_scat(j % 4, j % 8)

    plsc.subcore_barrier()
    for c in range(4):
        r0 = sid * ROWS_PER_TILE + c * RCHUNK
        pltpu.sync_copy(acc.at[pl.ds(r0, RCHUNK)], stg_v)
        pltpu.sync_copy(stg_v, out_hbm.at[cid].at[pl.ds(r0, RCHUNK)])


def _make_agg(dcols, nr):
    nq = 2 * nr
    pf = nr + 1

    @functools.partial(
        pl.kernel,
        out_type=jax.ShapeDtypeStruct((NC, NP, dcols), _F32),
        mesh=_MESH,
        scratch_types=[
            pltpu.VMEM((nq, 2, EBLK), jnp.int32),
            pltpu.VMEM((nr, EBLK, dcols), _F32),
            pltpu.VMEM_SHARED((NP, dcols), _F32),
            [pltpu.SemaphoreType.DMA] * nq,
            [pltpu.SemaphoreType.DMA] * nr,
            [pltpu.SemaphoreType.DMA] * nr,
        ],
        compiler_params=_SC_PARAMS,
    )
    def agg(hs_hbm, idx3_hbm, out_hbm, idx_v, rows_v, acc, isem, gsem, ssem):
        cid = lax.axis_index("c")
        sid = lax.axis_index("s")
        wid = cid * NS + sid
        base = wid * BLKS_PER_TILE
        LAST = BLKS_PER_TILE - 1

        def fire_idx(q, blk):
            pltpu.async_copy(idx3_hbm.at[blk], idx_v.at[q], isem[q])

        def drain_idx(q):
            pltpu.make_async_copy(
                idx3_hbm.at[base], idx_v.at[q], isem[q]).wait()

        def fire_gather(g, q):
            pltpu.async_copy(
                hs_hbm.at[idx_v.at[q, 0]], rows_v.at[g], gsem[g])

        def drain_gather(g, q):
            pltpu.make_async_copy(
                hs_hbm.at[idx_v.at[q, 0]], rows_v.at[g], gsem[g]).wait()

        def fire_scat(g, q):
            pltpu.async_copy(
                rows_v.at[g], acc.at[idx_v.at[q, 1]], ssem[g], add=True)

        def drain_scat(g, q):
            pltpu.make_async_copy(
                rows_v.at[g], acc.at[idx_v.at[q, 1]], ssem[g]).wait()

        def turn(jmod, blk, scat_on=True, idx_on=True, gat_on=True):
            g, q = jmod % nr, jmod % nq
            drain_gather(g, q)
            fire_scat(g, q)
            if scat_on:
                drain_scat((jmod + 1) % nr, (jmod + 1 - nr) % nq)
            if idx_on:
                fire_idx((jmod + pf) % nq, blk + pf)
            if gat_on:
                drain_idx((jmod + 1) % nq)
                fire_gather((jmod + 1) % nr, (jmod + 1) % nq)

        zeros16 = jnp.zeros((16,), _F32)

        @pl.loop(0, EBLK)
        def _(i):
            for k in range(dcols // 16):
                rows_v[0, i, pl.ds(k * 16, 16)] = zeros16

        for c in range(ROWS_PER_TILE // EBLK):
            pltpu.sync_copy(
                rows_v.at[0],
                acc.at[pl.ds(sid * ROWS_PER_TILE + c * EBLK, EBLK)])
        plsc.subcore_barrier()

        for q in range(pf):
            fire_idx(q, base + q)
        drain_idx(0)
        fire_gather(0, 0)
        for j in range(nq):
            turn(j, base + j, scat_on=(j + 1 - nr >= 0))

        @pl.loop(0, (BLKS_PER_TILE - 2 * nq) // nq)
        def _(i):
            for t in range(nq):
                turn(t, base + nq + nq * i + t)

        for j in range(BLKS_PER_TILE - nq, BLKS_PER_TILE):
            turn(j, base + j,
                 idx_on=(j + pf <= LAST), gat_on=(j + 1 <= LAST))
        for j in range(BLKS_PER_TILE - nr + 1, BLKS_PER_TILE):
            drain_scat(j % nr, j % nq)

        plsc.subcore_barrier()
        for c in range(ROWS_PER_TILE // EBLK):
            r0 = sid * ROWS_PER_TILE + c * EBLK
            pltpu.sync_copy(acc.at[pl.ds(r0, EBLK)], rows_v.at[0])
            pltpu.sync_copy(rows_v.at[0], out_hbm.at[cid].at[pl.ds(r0, EBLK)])

    return agg


_agg128 = _make_agg(H1, 2)
_agg64 = _make_agg(GDIM, 4)


def _dinv_from(deg_ref):
    deg = (deg_ref[0, pl.ds(0, NN), 0:1] + deg_ref[1, pl.ds(0, NN), 0:1]
           + 1.0)
    return lax.rsqrt(deg)


def _scale1_body(deg_ref, x_ref, w1_ref, hs_ref):
    dinv = _dinv_from(deg_ref)
    h = jnp.dot(x_ref[...], w1_ref[...], precision=_PREC,
                preferred_element_type=_F32)
    hs_ref[pl.ds(0, NN), :] = h * dinv
    hs_ref[pl.ds(NN, NP - NN), :] = jnp.zeros((NP - NN, H1), _F32)


def _layer2_body(p_ref, hs1_ref, deg_ref, w2_ref, b1_ref, hs2_ref):
    dinv = _dinv_from(deg_ref)
    agg = (p_ref[0, pl.ds(0, NN), :] + p_ref[1, pl.ds(0, NN), :]
           + hs1_ref[pl.ds(0, NN), :])
    out1 = jnp.maximum(agg * dinv + b1_ref[...], 0.0)
    h2 = jnp.dot(out1, w2_ref[...], precision=_PREC,
                 preferred_element_type=_F32)
    hs2_ref[pl.ds(0, NN), :] = h2 * dinv
    hs2_ref[pl.ds(NN, NP - NN), :] = jnp.zeros((NP - NN, GDIM), _F32)


def _head_body(q_ref, hs2_ref, deg_ref, b2_ref, batch_ref, img_ref,
               wi1_ref, bi1_ref, wi2_ref, bi2_ref, wc1_ref, bc1_ref,
               gamma_ref, beta_ref, wc2_ref, bc2_ref, out_ref):
    dinv = _dinv_from(deg_ref)
    out2 = ((q_ref[0, pl.ds(0, NN), :] + q_ref[1, pl.ds(0, NN), :]
             + hs2_ref[pl.ds(0, NN), :]) * dinv + b2_ref[...])
    iota = lax.broadcasted_iota(jnp.int32, (NN, BB), 1)
    oh = (batch_ref[...] == iota).astype(_F32)
    sums = lax.dot_general(oh, out2, (((0,), (0,)), ((), ())),
                           precision=_PREC, preferred_element_type=_F32)
    counts = jnp.sum(oh, axis=0)[:, None]
    ge = sums / jnp.maximum(counts, 1.0)
    img = jnp.maximum(
        jnp.dot(img_ref[...], wi1_ref[...], precision=_PREC,
                preferred_element_type=_F32) + bi1_ref[...], 0.0)
    ie = jnp.dot(img, wi2_ref[...], precision=_PREC,
                 preferred_element_type=_F32) + bi2_ref[...]
    comb = jnp.concatenate([ge, ie], axis=1)
    z = jnp.maximum(
        jnp.dot(comb, wc1_ref[...], precision=_PREC,
                preferred_element_type=_F32) + bc1_ref[...], 0.0)
    z = z * (gamma_ref[...] * (1.0 / (1.0 + BN_EPS) ** 0.5)) + beta_ref[...]
    out_ref[...] = jnp.dot(z, wc2_ref[...], precision=_PREC,
                           preferred_element_type=_F32) + bc2_ref[...]


_scale1 = pl.pallas_call(
    _scale1_body, out_shape=jax.ShapeDtypeStruct((NP, H1), _F32))
_layer2 = pl.pallas_call(
    _layer2_body, out_shape=jax.ShapeDtypeStruct((NP, GDIM), _F32))
_head = pl.pallas_call(
    _head_body, out_shape=jax.ShapeDtypeStruct((BB, 1), _F32))


def kernel(x, edge_index, batch, image_features, W1, b1, W2, b2,
           Wi1, bi1, Wi2, bi2, Wc1, bc1, gamma, beta, Wc2, bc2):
    fill = NN + (jnp.arange(EP - EE, dtype=jnp.int32) % (NP - NN))
    src_p = jnp.concatenate([edge_index[0], fill])
    dst_p = jnp.concatenate([edge_index[1], fill])

    idx3 = jnp.stack(
        [src_p.reshape(-1, EBLK), dst_p.reshape(-1, EBLK)], axis=1)

    degp = _deg_kernel(dst_p)
    hs1 = _scale1(degp, x, W1)
    p1 = _agg128(hs1, idx3)
    hs2 = _layer2(p1, hs1, degp, W2, b1[None, :])
    p2 = _agg64(hs2, idx3)
    out = _head(p2, hs2, degp, b2[None, :], batch[:, None], image_features,
                Wi1, bi1[None, :], Wi2, bi2[None, :], Wc1, bc1[None, :],
                gamma[None, :], beta[None, :], Wc2, bc2[None, :])
    return out


# speedup vs baseline: 2.7438x; 1.0041x over previous
import functools

import jax
import jax.numpy as jnp
from jax import lax
from jax.experimental import pallas as pl
from jax.experimental.pallas import tpu as pltpu
from jax.experimental.pallas import tpu_sc as plsc

NN = 10000
NP = 10240
EE = 320000
F_IN = 128
H1 = 128
GDIM = 64
BB = 64
IMG = 1280
BN_EPS = 1e-5

NC = 2
NS = 16
NW = NC * NS
EBLK = 128
BLKS_PER_TILE = 80
EPT = BLKS_PER_TILE * EBLK
EP = EPT * NW
ROWS_PER_TILE = NP // NS
RCHUNK = ROWS_PER_TILE // 4

_MESH = plsc.VectorSubcoreMesh(
    core_axis_name="c", subcore_axis_name="s", num_cores=NC, num_subcores=NS)

_F32 = jnp.float32
_PREC = None
_SC_PARAMS = pltpu.CompilerParams(use_tc_tiling_on_sc=False)


def _zero_stage(stg_v, dcols):
    zeros16 = jnp.zeros((16,), _F32)

    @pl.loop(0, RCHUNK)
    def _(i):
        for k in range(dcols // 16):
            stg_v[i, pl.ds(k * 16, 16)] = zeros16


@functools.partial(
    pl.kernel,
    out_type=jax.ShapeDtypeStruct((NC, NP, 16), _F32),
    mesh=_MESH,
    scratch_types=[
        pltpu.VMEM((8, EBLK), jnp.int32),
        pltpu.VMEM((EBLK, 16), _F32),
        pltpu.VMEM((RCHUNK, 16), _F32),
        pltpu.VMEM_SHARED((NP, 16), _F32),
        [pltpu.SemaphoreType.DMA] * 8,
        [pltpu.SemaphoreType.DMA] * 4,
    ],
    compiler_params=_SC_PARAMS,
)
def _deg_kernel(dst_hbm, out_hbm, dst_v, ones_v, stg_v, acc, isem, ssem):
    cid = lax.axis_index("c")
    sid = lax.axis_index("s")
    wid = cid * NS + sid
    ones16 = jnp.ones((16,), _F32)
    base = wid * BLKS_PER_TILE
    LAST = BLKS_PER_TILE - 1

    def fire_idx(q, blk):
        off = pl.multiple_of(blk * EBLK, EBLK)
        pltpu.async_copy(dst_hbm.at[pl.ds(off, EBLK)], dst_v.at[q], isem[q])

    def drain_idx(q):
        pltpu.make_async_copy(
            dst_hbm.at[pl.ds(0, EBLK)], dst_v.at[q], isem[q]).wait()

    def fire_scat(s, q):
        pltpu.async_copy(ones_v, acc.at[dst_v.at[q]], ssem[s], add=True)

    def drain_scat(s, q):
        pltpu.make_async_copy(ones_v, acc.at[dst_v.at[q]], ssem[s]).wait()

    def turn(jmod, blk, scat_on=True, idx_on=True):
        q, s = jmod % 8, jmod % 4
        drain_idx(q)
        fire_scat(s, q)
        if scat_on:
            drain_scat((jmod + 1) % 4, (jmod - 3) % 8)
        if idx_on:
            fire_idx((jmod + 5) % 8, blk + 5)

    @pl.loop(0, EBLK)
    def _(i):
        ones_v[i] = ones16

    _zero_stage(stg_v, 16)
    for c in range(4):
        pltpu.sync_copy(
            stg_v, acc.at[pl.ds(sid * ROWS_PER_TILE + c * RCHUNK, RCHUNK)])
    plsc.subcore_barrier()

    for q in range(5):
        fire_idx(q, base + q)
    for j in range(8):
        turn(j, base + j, scat_on=(j - 3 >= 0))

    @pl.loop(0, (BLKS_PER_TILE - 16) // 8)
    def _(i):
        for t in range(8):
            turn(t, base + 8 + 8 * i + t)

    for j in range(BLKS_PER_TILE - 8, BLKS_PER_TILE):
        turn(j, base + j, idx_on=(j + 5 <= LAST))
    for j in range(BLKS_PER_TILE - 3, BLKS_PER_TILE):
        drain_scat(j % 4, j % 8)

    plsc.subcore_barrier()
    for c in range(4):
        r0 = sid * ROWS_PER_TILE + c * RCHUNK
        pltpu.sync_copy(acc.at[pl.ds(r0, RCHUNK)], stg_v)
        pltpu.sync_copy(stg_v, out_hbm.at[cid].at[pl.ds(r0, RCHUNK)])


def _make_agg(dcols, nr):
    nq = 2 * nr
    pf = nr + 1

    @functools.partial(
        pl.kernel,
        out_type=jax.ShapeDtypeStruct((NC, NP, dcols), _F32),
        mesh=_MESH,
        scratch_types=[
            pltpu.VMEM((nq, 2, EBLK), jnp.int32),
            pltpu.VMEM((nr, EBLK, dcols), _F32),
            pltpu.VMEM_SHARED((NP, dcols), _F32),
            [pltpu.SemaphoreType.DMA] * nq,
            [pltpu.SemaphoreType.DMA] * nr,
            [pltpu.SemaphoreType.DMA] * nr,
        ],
        compiler_params=_SC_PARAMS,
    )
    def agg(hs_hbm, idx3_hbm, out_hbm, idx_v, rows_v, acc, isem, gsem, ssem):
        cid = lax.axis_index("c")
        sid = lax.axis_index("s")
        wid = cid * NS + sid
        base = wid * BLKS_PER_TILE
        LAST = BLKS_PER_TILE - 1

        def fire_idx(q, blk):
            pltpu.async_copy(idx3_hbm.at[blk], idx_v.at[q], isem[q])

        def drain_idx(q):
            pltpu.make_async_copy(
                idx3_hbm.at[base], idx_v.at[q], isem[q]).wait()

        def fire_gather(g, q):
            pltpu.async_copy(
                hs_hbm.at[idx_v.at[q, 0]], rows_v.at[g], gsem[g])

        def drain_gather(g, q):
            pltpu.make_async_copy(
                hs_hbm.at[idx_v.at[q, 0]], rows_v.at[g], gsem[g]).wait()

        def fire_scat(g, q):
            pltpu.async_copy(
                rows_v.at[g], acc.at[idx_v.at[q, 1]], ssem[g], add=True)

        def drain_scat(g, q):
            pltpu.make_async_copy(
                rows_v.at[g], acc.at[idx_v.at[q, 1]], ssem[g]).wait()

        def turn(jmod, blk, scat_on=True, idx_on=True, gat_on=True):
            g, q = jmod % nr, jmod % nq
            drain_gather(g, q)
            fire_scat(g, q)
            if scat_on:
                drain_scat((jmod + 1) % nr, (jmod + 1 - nr) % nq)
            if idx_on:
                fire_idx((jmod + pf) % nq, blk + pf)
            if gat_on:
                drain_idx((jmod + 1) % nq)
                fire_gather((jmod + 1) % nr, (jmod + 1) % nq)

        zeros16 = jnp.zeros((16,), _F32)

        @pl.loop(0, EBLK)
        def _(i):
            for k in range(dcols // 16):
                rows_v[0, i, pl.ds(k * 16, 16)] = zeros16

        for c in range(ROWS_PER_TILE // EBLK):
            pltpu.sync_copy(
                rows_v.at[0],
                acc.at[pl.ds(sid * ROWS_PER_TILE + c * EBLK, EBLK)])
        plsc.subcore_barrier()

        for q in range(pf):
            fire_idx(q, base + q)
        drain_idx(0)
        fire_gather(0, 0)
        for j in range(nq):
            turn(j, base + j, scat_on=(j + 1 - nr >= 0))

        @pl.loop(0, (BLKS_PER_TILE - 2 * nq) // nq)
        def _(i):
            for t in range(nq):
                turn(t, base + nq + nq * i + t)

        for j in range(BLKS_PER_TILE - nq, BLKS_PER_TILE):
            turn(j, base + j,
                 idx_on=(j + pf <= LAST), gat_on=(j + 1 <= LAST))
        for j in range(BLKS_PER_TILE - nr + 1, BLKS_PER_TILE):
            drain_scat(j % nr, j % nq)

        plsc.subcore_barrier()
        for c in range(ROWS_PER_TILE // EBLK):
            r0 = sid * ROWS_PER_TILE + c * EBLK
            pltpu.sync_copy(acc.at[pl.ds(r0, EBLK)], rows_v.at[0])
            pltpu.sync_copy(rows_v.at[0], out_hbm.at[cid].at[pl.ds(r0, EBLK)])

    return agg


_agg128 = _make_agg(H1, 2)
_agg64 = _make_agg(GDIM, 4)


def _dinv_from(deg_ref):
    deg = (deg_ref[0, pl.ds(0, NN), 0:1] + deg_ref[1, pl.ds(0, NN), 0:1]
           + 1.0)
    return lax.rsqrt(deg)


def _scale1_body(deg_ref, x_ref, w1_ref, hs_ref):
    dinv = _dinv_from(deg_ref)
    h = jnp.dot(x_ref[...], w1_ref[...], precision=_PREC,
                preferred_element_type=_F32)
    hs_ref[pl.ds(0, NN), :] = h * dinv
    hs_ref[pl.ds(NN, NP - NN), :] = jnp.zeros((NP - NN, H1), _F32)


def _layer2_body(p_ref, hs1_ref, deg_ref, w2_ref, b1_ref, hs2_ref):
    dinv = _dinv_from(deg_ref)
    agg = (p_ref[0, pl.ds(0, NN), :] + p_ref[1, pl.ds(0, NN), :]
           + hs1_ref[pl.ds(0, NN), :])
    out1 = jnp.maximum(agg * dinv + b1_ref[...], 0.0)
    h2 = jnp.dot(out1, w2_ref[...], precision=_PREC,
                 preferred_element_type=_F32)
    hs2_ref[pl.ds(0, NN), :] = h2 * dinv
    hs2_ref[pl.ds(NN, NP - NN), :] = jnp.zeros((NP - NN, GDIM), _F32)


def _head_body(q_ref, hs2_ref, deg_ref, b2_ref, batch_ref, img_ref,
               wi1_ref, bi1_ref, wi2_ref, bi2_ref, wc1_ref, bc1_ref,
               gamma_ref, beta_ref, wc2_ref, bc2_ref, out_ref):
    dinv = _dinv_from(deg_ref)
    out2 = ((q_ref[0, pl.ds(0, NN), :] + q_ref[1, pl.ds(0, NN), :]
             + hs2_ref[pl.ds(0, NN), :]) * dinv + b2_ref[...])
    iota = lax.broadcasted_iota(jnp.int32, (NN, BB), 1)
    oh = (batch_ref[...] == iota).astype(_F32)
    sums = lax.dot_general(oh, out2, (((0,), (0,)), ((), ())),
                           precision=_PREC, preferred_element_type=_F32)
    counts = jnp.sum(oh, axis=0)[:, None]
    ge = sums / jnp.maximum(counts, 1.0)
    img = jnp.maximum(
        jnp.dot(img_ref[...], wi1_ref[...], precision=_PREC,
                preferred_element_type=_F32) + bi1_ref[...], 0.0)
    ie = jnp.dot(img, wi2_ref[...], precision=_PREC,
                 preferred_element_type=_F32) + bi2_ref[...]
    comb = jnp.concatenate([ge, ie], axis=1)
    z = jnp.maximum(
        jnp.dot(comb, wc1_ref[...], precision=_PREC,
                preferred_element_type=_F32) + bc1_ref[...], 0.0)
    z = z * (gamma_ref[...] * (1.0 / (1.0 + BN_EPS) ** 0.5)) + beta_ref[...]
    out_ref[...] = jnp.dot(z, wc2_ref[...], precision=_PREC,
                           preferred_element_type=_F32) + bc2_ref[...]


_scale1 = pl.pallas_call(
    _scale1_body, out_shape=jax.ShapeDtypeStruct((NP, H1), _F32))
_layer2 = pl.pallas_call(
    _layer2_body, out_shape=jax.ShapeDtypeStruct((NP, GDIM), _F32))
_head = pl.pallas_call(
    _head_body, out_shape=jax.ShapeDtypeStruct((BB, 1), _F32))


def kernel(x, edge_index, batch, image_features, W1, b1, W2, b2,
           Wi1, bi1, Wi2, bi2, Wc1, bc1, gamma, beta, Wc2, bc2):
    fill = NN + (jnp.arange(EP - EE, dtype=jnp.int32) % (NP - NN))
    src_p = jnp.concatenate([edge_index[0], fill])
    dst_p = jnp.concatenate([edge_index[1], fill])

    idx3 = jnp.stack(
        [src_p.reshape(-1, EBLK), dst_p.reshape(-1, EBLK)], axis=1)

    degp = _deg_kernel(dst_p)
    hs1 = _scale1(degp, x, W1)
    p1 = _agg128(hs1, idx3)
    hs2 = _layer2(p1, hs1, degp, W2, b1[None, :])
    p2 = _agg64(hs2, idx3)
    out = _head(p2, hs2, degp, b2[None, :], batch[:, None], image_features,
                Wi1, bi1[None, :], Wi2, bi2[None, :], Wc1, bc1[None, :],
                gamma[None, :], beta[None, :], Wc2, bc2[None, :])
    return out
